# Initial kernel scaffold; baseline (speedup 1.0000x reference)
import jax
import jax.numpy as jnp
from jax.experimental import pallas as pl


def kernel(ab_pairs, values, mask):
    raise NotImplementedError("write your pallas kernel here")


# speedup vs baseline: 1.3108x; 1.3108x over previous
import jax
import jax.numpy as jnp
from jax.experimental import pallas as pl
from jax.experimental.pallas import tpu as pltpu

_SAMPLING_FRACTION = 0.25
_INIT_DIST = 100000000.0
_SENTINEL = -1.0e9


def _fps_body(f0_ref, ab_ref, vals_ref, subab_ref, subv_ref, rows_ref,
              rows_t_ref, q_ref):
    b = pl.program_id(0)
    nf = ab_ref.shape[2]
    n_samples = subv_ref.shape[1]

    lane = jax.lax.broadcasted_iota(jnp.int32, (1, nf), 1)
    is_mid = (lane % 3) == 1
    dist0 = jnp.where(is_mid, jnp.float32(_INIT_DIST), jnp.float32(_SENTINEL))
    f0 = f0_ref[b]

    def step(t, carry):
        dist, f = carry
        q_ref[t] = f
        row = ab_ref[0, pl.ds(f, 1), :]
        rows_ref[pl.ds(t, 1), :] = row
        subv_ref[0, pl.ds(t, 1), :] = vals_ref[0, pl.ds(f, 1), :]
        s = row * row
        y = (s + pltpu.roll(s, 1, 1)) + pltpu.roll(s, nf - 1, 1)
        d = jnp.sqrt(y)
        dist = jnp.minimum(dist, jnp.where(is_mid, d, jnp.float32(_SENTINEL)))
        kstar = jnp.argmax(dist, axis=1)[0]
        f_new = (kstar // 3).astype(jnp.int32)
        return dist, f_new

    jax.lax.fori_loop(0, n_samples, step, (dist0, f0))

    for c in range(nf // 128):
        rows_t_ref[c * 128:(c + 1) * 128, :] = jnp.swapaxes(
            rows_ref[:, c * 128:(c + 1) * 128], 0, 1)

    def gather_col(u, _):
        qu = q_ref[u]
        for d in range(3):
            subab_ref[0, pl.ds(d * n_samples + u, 1), :] = (
                rows_t_ref[pl.ds(3 * qu + d, 1), :])
        return 0

    jax.lax.fori_loop(0, n_samples, gather_col, 0)


def kernel(ab_pairs, values, mask):
    B, N = mask.shape
    D = ab_pairs.shape[-1]
    V = values.shape[-1]
    S = int(round(_SAMPLING_FRACTION * N))
    NF = N * D

    key = jax.random.key(42)
    rand_idx = jax.random.randint(key, (B,), 0, N)
    counts = mask.sum(-1)
    tmp = rand_idx % counts
    csum = jnp.cumsum(mask.astype(jnp.int32), axis=-1)
    f0 = jnp.argmax((csum == (tmp[:, None] + 1)) & mask, axis=-1).astype(jnp.int32)

    ab_flat = ab_pairs.reshape(B, N, NF)

    sub_ab_udt, sub_vals = pl.pallas_call(
        _fps_body,
        grid=(B,),
        in_specs=[
            pl.BlockSpec(memory_space=pltpu.SMEM),
            pl.BlockSpec((1, N, NF), lambda b: (b, 0, 0)),
            pl.BlockSpec((1, N, V), lambda b: (b, 0, 0)),
        ],
        out_specs=[
            pl.BlockSpec((1, S * D, S), lambda b: (b, 0, 0)),
            pl.BlockSpec((1, S, V), lambda b: (b, 0, 0)),
        ],
        out_shape=[
            jax.ShapeDtypeStruct((B, S * D, S), jnp.float32),
            jax.ShapeDtypeStruct((B, S, V), jnp.float32),
        ],
        scratch_shapes=[
            pltpu.VMEM((S, NF), jnp.float32),
            pltpu.VMEM((NF, S), jnp.float32),
            pltpu.SMEM((S,), jnp.int32),
        ],
        compiler_params=pltpu.CompilerParams(
            dimension_semantics=("arbitrary",),
        ),
    )(f0, ab_flat, values)

    sub_ab = jnp.transpose(sub_ab_udt.reshape(B, D, S, S), (0, 2, 3, 1))
    sub_mask = jnp.ones((B, S), dtype=mask.dtype) & jnp.all(
        mask, axis=1, keepdims=True
    )
    return sub_ab, sub_vals, sub_mask
